# baseline (speedup 1.0000x reference)
import functools
import math

import jax
import jax.numpy as jnp
from jax import lax
from jax.experimental import pallas as pl
from jax.experimental.pallas import tpu as pltpu
from jax.experimental.pallas import tpu_sc as plsc

B = 16384
NC, NS = 2, 16
NW = NC * NS
BPW = B // NW
CHUNK = 128
NCH = BPW // CHUNK

D_ITEM, D_CAT, D_PRP = 32, 16, 16
H, OUT = 256, 64
_BN = 1.0 / math.sqrt(1.0 + 1e-5)

_TABLE_DIMS = (D_ITEM, D_CAT, D_CAT, D_CAT, D_PRP)

_sc_mesh = plsc.VectorSubcoreMesh(
    core_axis_name="c", subcore_axis_name="s", num_cores=NC, num_subcores=NS)


def _sc_gather_body(c0, c1, c2, c3, c4, t0, t1, t2, t3, t4,
                    e0, e1, e2, e3, e4,
                    i0, i1, i2, i3, i4,
                    r0, r1, r2, r3, r4,
                    s0, s1, s2, s3, s4):
    wid = lax.axis_index("s") * NC + lax.axis_index("c")
    base = wid * BPW
    idx_refs = (i0, i1, i2, i3, i4)
    row_refs = (r0, r1, r2, r3, r4)
    tables = (t0, t1, t2, t3, t4)
    outs = (e0, e1, e2, e3, e4)
    sems = (s0, s1, s2, s3, s4)

    for cref, iref in zip((c0, c1, c2, c3, c4), idx_refs):
        pltpu.sync_copy(cref.at[pl.ds(wid * NCH, NCH)], iref)

    handles = []
    for t in range(5):
        for j in range(NCH):
            handles.append(pltpu.async_copy(
                tables[t].at[idx_refs[t].at[j]],
                row_refs[t].at[pl.ds(j * CHUNK, CHUNK)],
                sems[t]))
    for h in handles:
        h.wait()
    for t in range(5):
        pltpu.sync_copy(row_refs[t], outs[t].at[pl.ds(base, BPW)])


_sc_gather = pl.kernel(
    _sc_gather_body,
    out_type=[jax.ShapeDtypeStruct((B, d), jnp.float32) for d in _TABLE_DIMS],
    mesh=_sc_mesh,
    scratch_types=(
        [pltpu.VMEM((NCH, CHUNK), jnp.int32) for _ in range(5)]
        + [pltpu.VMEM((BPW, d), jnp.float32) for d in _TABLE_DIMS]
        + [pltpu.SemaphoreType.DMA for _ in range(5)]),
    compiler_params=pltpu.CompilerParams(use_tc_tiling_on_sc=False),
)


def _mlp_body(e0, e1, e2, e3, e4, dn, w1a, w1b, w1c, w1d, w1e, w1f,
              b1, gm, bt, w2, b2, out):
    h = jnp.dot(e0[...], w1a[...], preferred_element_type=jnp.float32)
    h = h + jnp.dot(e1[...], w1b[...], preferred_element_type=jnp.float32)
    h = h + jnp.dot(e2[...], w1c[...], preferred_element_type=jnp.float32)
    h = h + jnp.dot(e3[...], w1d[...], preferred_element_type=jnp.float32)
    h = h + jnp.dot(e4[...], w1e[...], preferred_element_type=jnp.float32)
    h = h + jnp.dot(dn[...], w1f[...], preferred_element_type=jnp.float32)
    h = (h + b1[...]) * (_BN * gm[...]) + bt[...]
    h = jnp.maximum(h, 0.0)
    o = jnp.dot(h, w2[...], preferred_element_type=jnp.float32) + b2[...]
    nrm = jnp.sqrt(jnp.sum(o * o, axis=1, keepdims=True))
    out[...] = o / jnp.maximum(nrm, 1e-12)


def _mlp(e0, e1, e2, e3, e4, dn, w1a, w1b, w1c, w1d, w1e, w1f,
         b1, gm, bt, w2, b2, block_rows=2048):
    grid = (B // block_rows,)

    def row_spec(d):
        return pl.BlockSpec((block_rows, d), lambda i: (i, 0))

    def full_spec(shape):
        return pl.BlockSpec(shape, lambda i: (0, 0))

    return pl.pallas_call(
        _mlp_body,
        grid=grid,
        in_specs=[
            row_spec(D_ITEM), row_spec(D_CAT), row_spec(D_CAT),
            row_spec(D_CAT), row_spec(D_PRP), row_spec(3),
            full_spec((D_ITEM, H)), full_spec((D_CAT, H)),
            full_spec((D_CAT, H)), full_spec((D_CAT, H)),
            full_spec((D_PRP, H)), full_spec((3, H)),
            full_spec((1, H)), full_spec((1, H)), full_spec((1, H)),
            full_spec((H, OUT)), full_spec((1, OUT)),
        ],
        out_specs=pl.BlockSpec((block_rows, OUT), lambda i: (i, 0)),
        out_shape=jax.ShapeDtypeStruct((B, OUT), jnp.float32),
    )(e0, e1, e2, e3, e4, dn, w1a, w1b, w1c, w1d, w1e, w1f,
      b1, gm, bt, w2, b2)


def kernel(item_cat, item_dense, item_emb, cat_l1_emb, cat_l2_emb,
           brand_emb, price_emb, W1, b1, gamma, beta, W2, b2):
    ic = item_cat.astype(jnp.int32)
    cols = [ic[:, j].reshape(B // CHUNK, CHUNK) for j in range(5)]
    pr_pad = jnp.pad(price_emb, ((0, 0), (0, D_PRP - price_emb.shape[1])))

    e0, e1, e2, e3, e4 = _sc_gather(
        *cols, item_emb, cat_l1_emb, cat_l2_emb, brand_emb, pr_pad)

    w1a = W1[0:32]
    w1b = W1[32:48]
    w1c = W1[48:64]
    w1d = W1[64:80]
    w1e = jnp.pad(W1[80:88], ((0, D_PRP - 8), (0, 0)))
    w1f = W1[88:91]

    return _mlp(e0, e1, e2, e3, e4, item_dense,
                w1a, w1b, w1c, w1d, w1e, w1f,
                b1.reshape(1, H), gamma.reshape(1, H), beta.reshape(1, H),
                W2, b2.reshape(1, OUT))
